# baseline (speedup 1.0000x reference)
import jax
import jax.numpy as jnp
from jax.experimental import pallas as pl


def _ffn_kernel(x_ref, w1_ref, b1_ref, w2_ref, b2_ref, o_ref):
    i = pl.program_id(1)
    x = x_ref[0]
    w1 = w1_ref[0]
    h = jax.lax.dot_general(
        x, w1, (((1,), (1,)), ((), ())), preferred_element_type=jnp.float32
    )
    h = h + b1_ref[0]
    h = 0.5 * h * (1.0 + jax.lax.erf(h * 0.7071067811865476))
    w2 = w2_ref[0]
    y = jax.lax.dot_general(
        h, w2, (((1,), (1,)), ((), ())), preferred_element_type=jnp.float32
    )

    @pl.when(i == 0)
    def _init():
        o_ref[0] = y + b2_ref[0]

    @pl.when(i > 0)
    def _acc():
        o_ref[0] = o_ref[0] + y


def kernel(inter_state, W1, b1, W2, b2, loads):
    E, I, H = W1.shape
    CAP = inter_state.shape[0] // E
    TI = 768
    NI = I // TI

    x = inter_state.reshape(E, CAP, H)
    b1r = b1.reshape(E, 1, I)
    b2r = b2.reshape(E, 1, H)

    out = pl.pallas_call(
        _ffn_kernel,
        grid=(E, NI),
        in_specs=[
            pl.BlockSpec((1, CAP, H), lambda e, i: (e, 0, 0)),
            pl.BlockSpec((1, TI, H), lambda e, i: (e, i, 0)),
            pl.BlockSpec((1, 1, TI), lambda e, i: (e, 0, i)),
            pl.BlockSpec((1, H, TI), lambda e, i: (e, 0, i)),
            pl.BlockSpec((1, 1, H), lambda e, i: (e, 0, 0)),
        ],
        out_specs=pl.BlockSpec((1, CAP, H), lambda e, i: (e, 0, 0)),
        out_shape=jax.ShapeDtypeStruct((E, CAP, H), jnp.float32),
    )(x, W1, b1r, W2, b2r)
    return out.reshape(E * CAP, H)


# speedup vs baseline: 1.2335x; 1.2335x over previous
import jax
import jax.numpy as jnp
from jax.experimental import pallas as pl
from jax.experimental.pallas import tpu as pltpu


def _ffn_kernel(x_ref, w1_ref, b1_ref, w2_ref, b2_ref, o_ref):
    x = x_ref[0]
    w1 = w1_ref[0]
    h = jax.lax.dot_general(
        x, w1, (((1,), (1,)), ((), ())), preferred_element_type=jnp.float32
    )
    h = h + b1_ref[0]
    h = 0.5 * h * (1.0 + jax.lax.erf(h * 0.7071067811865476))
    w2 = w2_ref[0]
    y = jax.lax.dot_general(
        h, w2, (((1,), (1,)), ((), ())), preferred_element_type=jnp.float32
    )
    o_ref[0] = y + b2_ref[0]


def kernel(inter_state, W1, b1, W2, b2, loads):
    E, I, H = W1.shape
    CAP = inter_state.shape[0] // E

    x = inter_state.reshape(E, CAP, H)
    b1r = b1.reshape(E, 1, I)
    b2r = b2.reshape(E, 1, H)

    out = pl.pallas_call(
        _ffn_kernel,
        grid=(E,),
        in_specs=[
            pl.BlockSpec((1, CAP, H), lambda e: (e, 0, 0)),
            pl.BlockSpec((1, I, H), lambda e: (e, 0, 0)),
            pl.BlockSpec((1, 1, I), lambda e: (e, 0, 0)),
            pl.BlockSpec((1, H, I), lambda e: (e, 0, 0)),
            pl.BlockSpec((1, 1, H), lambda e: (e, 0, 0)),
        ],
        out_specs=pl.BlockSpec((1, CAP, H), lambda e: (e, 0, 0)),
        out_shape=jax.ShapeDtypeStruct((E, CAP, H), jnp.float32),
        compiler_params=pltpu.CompilerParams(
            dimension_semantics=("parallel",),
        ),
    )(x, W1, b1r, W2, b2r)
    return out.reshape(E * CAP, H)
